# baseline (speedup 1.0000x reference)
import functools

import jax
import jax.numpy as jnp
from jax import lax
from jax.experimental import pallas as pl
from jax.experimental.pallas import tpu as pltpu
from jax.experimental.pallas import tpu_sc as plsc

V_NUM = 1000000
H_DIM = 32
BATCH = 16384
FIELDS = 26

_N = BATCH * FIELDS
_NW = 32
_PER_W = _N // _NW
_GRP = 128
_G_PER_W = _PER_W // _GRP
_GPC = 8
_CH = _GPC * _GRP
_NCH = _G_PER_W // _GPC
_NBUF = 3
_TCG = BATCH // _GRP


_VFULL = V_NUM // _GRP * _GRP
_NFG = _VFULL // _GRP
_VTAIL = V_NUM - _VFULL


def _make_detile_kernel():
    mesh = plsc.VectorSubcoreMesh(core_axis_name="c", subcore_axis_name="s")

    @functools.partial(
        pl.kernel,
        mesh=mesh,
        out_type=jax.ShapeDtypeStruct((V_NUM * H_DIM,), jnp.float32),
        compiler_params=pltpu.CompilerParams(
            use_tc_tiling_on_sc=True, needs_layout_passes=False),
        scratch_types=(
            pltpu.VMEM((2, H_DIM, 129), jnp.float32),
            pltpu.VMEM((2, _GRP * H_DIM), jnp.float32),
            pltpu.SemaphoreType.DMA((2,)),
            pltpu.SemaphoreType.DMA((2,)),
        ),
    )
    def detile_kernel(tab_t_hbm, out_hbm, tsrc, tdst, rsem, wsem):
        wid = lax.axis_index("s") * 2 + lax.axis_index("c")
        cnt = jnp.where(wid < 4, _NFG // _NW + 1, _NFG // _NW)
        iota = lax.iota(jnp.int32, 16)

        def r_copies(i, s):
            g = i * _NW + wid
            for tr in range(H_DIM // 8):
                yield pltpu.make_async_copy(
                    tab_t_hbm.at[pl.ds(8 * tr, 8), pl.ds(g * _GRP, _GRP)],
                    tsrc.at[s, pl.ds(8 * tr, 8), pl.ds(0, _GRP)],
                    rsem.at[s])

        def w_copy(i, s):
            g = i * _NW + wid
            return pltpu.make_async_copy(
                tdst.at[s],
                out_hbm.at[pl.ds(g * _GRP * H_DIM, _GRP * H_DIM)],
                wsem.at[s])

        for c in r_copies(0, 0):
            c.start()

        def body(i, _):
            s = lax.rem(i, 2)
            for c in r_copies(i, s):
                c.wait()

            @pl.when(i + 1 < cnt)
            def _():
                for c in r_copies(i + 1, 1 - s):
                    c.start()

            @pl.when(i >= 2)
            def _():
                w_copy(i - 2, s).wait()

            for l in range(_GRP):
                for k in range(H_DIM // 16):
                    vec = plsc.load_gather(
                        tsrc.at[s],
                        [16 * k + iota, jnp.full((16,), l, jnp.int32)])
                    tdst[s, pl.ds(l * H_DIM + 16 * k, 16)] = vec
            w_copy(i, s).start()
            return _

        lax.fori_loop(0, cnt, body, None, unroll=False)
        for back in range(2):
            @pl.when(cnt > back)
            def _():
                i = cnt - 1 - back
                w_copy(i, lax.rem(i, 2)).wait()

        @pl.when(wid == _NW - 1)
        def _():
            for c in range(H_DIM):
                pltpu.sync_copy(
                    tab_t_hbm.at[c, pl.ds(_VFULL, _VTAIL)],
                    tsrc.at[0, c, pl.ds(0, _VTAIL)])
            for l in range(_VTAIL):
                for k in range(H_DIM // 16):
                    vec = plsc.load_gather(
                        tsrc.at[0],
                        [16 * k + iota, jnp.full((16,), l, jnp.int32)])
                    tdst[0, pl.ds(l * H_DIM + 16 * k, 16)] = vec
            pltpu.sync_copy(
                tdst.at[0, pl.ds(0, _VTAIL * H_DIM)],
                out_hbm.at[pl.ds(_VFULL * H_DIM, _VTAIL * H_DIM)])

    return detile_kernel


def _make_kernel():
    mesh = plsc.VectorSubcoreMesh(core_axis_name="c", subcore_axis_name="s")

    @functools.partial(
        pl.kernel,
        mesh=mesh,
        out_type=jax.ShapeDtypeStruct((FIELDS, H_DIM // 8, _TCG, 8, _GRP),
                                      jnp.float32),
        compiler_params=pltpu.CompilerParams(
            use_tc_tiling_on_sc=False, needs_layout_passes=False),
        scratch_types=(
            pltpu.VMEM((_PER_W,), jnp.int32),
            pltpu.VMEM((_NBUF * _CH, H_DIM), jnp.float32),
            pltpu.VMEM((2, H_DIM, 129), jnp.float32),
            pltpu.SemaphoreType.DMA((_NBUF,)),
            pltpu.SemaphoreType.DMA((2,)),
        ),
    )
    def gather_kernel(idx_hbm, table_hbm, out_hbm, idx_v, rows, tbuf,
                      gsem, wsem):
        wid = lax.axis_index("s") * 2 + lax.axis_index("c")
        base = wid * _PER_W
        g_base = wid * _G_PER_W
        pltpu.sync_copy(idx_hbm.at[pl.ds(base, _PER_W)], idx_v)

        def g_start(ch, slot):
            pltpu.async_copy(
                table_hbm.at[idx_v.at[pl.ds(ch * _CH, _CH)]],
                rows.at[pl.ds(slot * _CH, _CH)],
                gsem.at[slot])

        def g_wait(ch, slot):
            pltpu.make_async_copy(
                table_hbm.at[idx_v.at[pl.ds(ch * _CH, _CH)]],
                rows.at[pl.ds(slot * _CH, _CH)],
                gsem.at[slot]).wait()

        def t_writes(gg, p, do_wait):
            f = gg // _TCG
            tc = gg % _TCG
            for tr in range(H_DIM // 8):
                cpy = pltpu.make_async_copy(
                    tbuf.at[p, pl.ds(8 * tr, 8), pl.ds(0, _GRP)],
                    out_hbm.at[f, tr, tc],
                    wsem.at[p])
                if do_wait:
                    cpy.wait()
                else:
                    cpy.start()

        iota = lax.iota(jnp.int32, 16)

        for slot in range(_NBUF):
            g_start(slot, slot)

        def chunk_body(ch, _):
            slot = lax.rem(ch, _NBUF)
            g_wait(ch, slot)

            def group_body(g8, _):
                gl = ch * _GPC + g8
                gg = g_base + gl
                p = lax.rem(gl, 2)

                @pl.when(gl >= 2)
                def _():
                    t_writes(gg - 2, p, True)

                row0 = slot * _CH + g8 * _GRP
                for l in range(_GRP):
                    for k in range(H_DIM // 16):
                        vec = rows[row0 + l, pl.ds(16 * k, 16)]
                        plsc.store_scatter(
                            tbuf.at[p],
                            [16 * k + iota, jnp.full((16,), l, jnp.int32)],
                            vec)
                t_writes(gg, p, False)
                return _

            lax.fori_loop(0, _GPC, group_body, None, unroll=False)

            @pl.when(ch + _NBUF < _NCH)
            def _():
                g_start(ch + _NBUF, slot)
            return _

        lax.fori_loop(0, _NCH, chunk_body, None, unroll=False)

        for tail in range(2):
            gl = _G_PER_W - 2 + tail
            t_writes(g_base + gl, gl % 2, True)

    return gather_kernel


_KERNEL = _make_kernel()
_DETILE = _make_detile_kernel()


@jax.jit
def kernel(x, table):
    idx = x.T.reshape(-1).astype(jnp.int32)
    lin_table = _DETILE(table.T).reshape(V_NUM, H_DIM)
    out5 = _KERNEL(idx, lin_table)
    return out5.transpose(2, 4, 0, 1, 3).reshape(BATCH, FIELDS, H_DIM)


# speedup vs baseline: 3.1109x; 3.1109x over previous
import functools

import jax
import jax.numpy as jnp
from jax import lax
from jax.experimental import pallas as pl
from jax.experimental.pallas import tpu as pltpu
from jax.experimental.pallas import tpu_sc as plsc

V_NUM = 1000000
H_DIM = 32
BATCH = 16384
FIELDS = 26

_N = BATCH * FIELDS
_NW = 32
_PER_W = _N // _NW
_GRP = 128
_G_PER_W = _PER_W // _GRP
_GPC = 8
_CH = _GPC * _GRP
_NCH = _G_PER_W // _GPC
_NBUF = 3
_TCG = BATCH // _GRP


_VFULL = V_NUM // _GRP * _GRP
_NFG = _VFULL // _GRP
_VTAIL = V_NUM - _VFULL


def _make_detile_kernel():
    mesh = plsc.VectorSubcoreMesh(core_axis_name="c", subcore_axis_name="s")

    @functools.partial(
        pl.kernel,
        mesh=mesh,
        out_type=jax.ShapeDtypeStruct((V_NUM * H_DIM,), jnp.float32),
        compiler_params=pltpu.CompilerParams(
            use_tc_tiling_on_sc=True, needs_layout_passes=False),
        scratch_types=(
            pltpu.VMEM((2 * H_DIM, _GRP), jnp.float32),
            pltpu.VMEM((2 * _GRP * H_DIM,), jnp.float32),
            pltpu.SemaphoreType.DMA((2,)),
            pltpu.SemaphoreType.DMA((2,)),
        ),
    )
    def detile_kernel(tab_t_hbm, tail_hbm, out_hbm, tsrc, tdst, rsem, wsem):
        wid = lax.axis_index("s") * 2 + lax.axis_index("c")
        cnt = jnp.where(wid < 4, _NFG // _NW + 1, _NFG // _NW)
        iota = lax.iota(jnp.int32, 16)
        rots = [lax.rem(iota + d, 16) for d in range(16)]
        rot32s = [r * H_DIM + iota for r in rots]

        def r_copies(i, s):
            g = i * _NW + wid
            for tr in range(H_DIM // 8):
                yield pltpu.make_async_copy(
                    tab_t_hbm.at[pl.ds(8 * tr, 8), pl.ds(g * _GRP, _GRP)],
                    tsrc.at[pl.ds(s * H_DIM + 8 * tr, 8)],
                    rsem.at[s])

        def transpose_block(s, l0, nl):
            for lb in range(0, nl, 16):
                for k in range(H_DIM // 16):
                    for d in range(16):
                        vec = plsc.load_gather(
                            tsrc,
                            [s * H_DIM + 16 * k + iota, l0 + lb + rots[d]])
                        plsc.store_scatter(
                            tdst,
                            [rot32s[d]
                             + (s * (_GRP * H_DIM)
                                + (l0 + lb) * H_DIM + 16 * k)],
                            vec)

        def w_copy(i, s):
            g = i * _NW + wid
            return pltpu.make_async_copy(
                tdst.at[pl.ds(s * (_GRP * H_DIM), _GRP * H_DIM)],
                out_hbm.at[pl.ds(g * _GRP * H_DIM, _GRP * H_DIM)],
                wsem.at[s])

        for c in r_copies(0, 0):
            c.start()

        def body(i, _):
            s = lax.rem(i, 2)
            for c in r_copies(i, s):
                c.wait()

            @pl.when(i + 1 < cnt)
            def _():
                for c in r_copies(i + 1, 1 - s):
                    c.start()

            @pl.when(i >= 2)
            def _():
                w_copy(i - 2, s).wait()

            transpose_block(s, 0, _GRP)
            w_copy(i, s).start()
            return _

        lax.fori_loop(0, cnt, body, None, unroll=False)
        for back in range(2):
            @pl.when(cnt > back)
            def _():
                i = cnt - 1 - back
                w_copy(i, lax.rem(i, 2)).wait()

        @pl.when(wid == _NW - 1)
        def _():
            pltpu.sync_copy(tail_hbm, tdst.at[pl.ds(0, _VTAIL * H_DIM)])
            pltpu.sync_copy(
                tdst.at[pl.ds(0, _VTAIL * H_DIM)],
                out_hbm.at[pl.ds(_VFULL * H_DIM, _VTAIL * H_DIM)])

    return detile_kernel


def _make_kernel():
    mesh = plsc.VectorSubcoreMesh(core_axis_name="c", subcore_axis_name="s")

    @functools.partial(
        pl.kernel,
        mesh=mesh,
        out_type=jax.ShapeDtypeStruct((FIELDS, H_DIM // 8, _TCG, 8, _GRP),
                                      jnp.float32),
        compiler_params=pltpu.CompilerParams(
            use_tc_tiling_on_sc=False, needs_layout_passes=False),
        scratch_types=(
            pltpu.VMEM((_PER_W,), jnp.int32),
            pltpu.VMEM((_NBUF * _CH, H_DIM), jnp.float32),
            pltpu.VMEM((2, H_DIM, 129), jnp.float32),
            pltpu.SemaphoreType.DMA((_NBUF,)),
            pltpu.SemaphoreType.DMA((2,)),
        ),
    )
    def gather_kernel(idx_hbm, table_hbm, out_hbm, idx_v, rows, tbuf,
                      gsem, wsem):
        wid = lax.axis_index("s") * 2 + lax.axis_index("c")
        base = wid * _PER_W
        g_base = wid * _G_PER_W
        pltpu.sync_copy(idx_hbm.at[pl.ds(base, _PER_W)], idx_v)

        def g_start(ch, slot):
            pltpu.async_copy(
                table_hbm.at[idx_v.at[pl.ds(ch * _CH, _CH)]],
                rows.at[pl.ds(slot * _CH, _CH)],
                gsem.at[slot])

        def g_wait(ch, slot):
            pltpu.make_async_copy(
                table_hbm.at[idx_v.at[pl.ds(ch * _CH, _CH)]],
                rows.at[pl.ds(slot * _CH, _CH)],
                gsem.at[slot]).wait()

        def t_writes(gg, p, do_wait):
            f = gg // _TCG
            tc = gg % _TCG
            for tr in range(H_DIM // 8):
                cpy = pltpu.make_async_copy(
                    tbuf.at[p, pl.ds(8 * tr, 8), pl.ds(0, _GRP)],
                    out_hbm.at[f, tr, tc],
                    wsem.at[p])
                if do_wait:
                    cpy.wait()
                else:
                    cpy.start()

        iota = lax.iota(jnp.int32, 16)

        for slot in range(_NBUF):
            g_start(slot, slot)

        def chunk_body(ch, _):
            slot = lax.rem(ch, _NBUF)
            g_wait(ch, slot)

            def group_body(g8, _):
                gl = ch * _GPC + g8
                gg = g_base + gl
                p = lax.rem(gl, 2)

                @pl.when(gl >= 2)
                def _():
                    t_writes(gg - 2, p, True)

                row0 = slot * _CH + g8 * _GRP
                for l in range(_GRP):
                    for k in range(H_DIM // 16):
                        vec = rows[row0 + l, pl.ds(16 * k, 16)]
                        plsc.store_scatter(
                            tbuf.at[p],
                            [16 * k + iota, jnp.full((16,), l, jnp.int32)],
                            vec)
                t_writes(gg, p, False)
                return _

            lax.fori_loop(0, _GPC, group_body, None, unroll=False)

            @pl.when(ch + _NBUF < _NCH)
            def _():
                g_start(ch + _NBUF, slot)
            return _

        lax.fori_loop(0, _NCH, chunk_body, None, unroll=False)

        for tail in range(2):
            gl = _G_PER_W - 2 + tail
            t_writes(g_base + gl, gl % 2, True)

    return gather_kernel


_KERNEL = _make_kernel()
_DETILE = _make_detile_kernel()


@jax.jit
def kernel(x, table):
    idx = x.T.reshape(-1).astype(jnp.int32)
    tail_lin = table[_VFULL:].reshape(-1)
    lin_table = _DETILE(table.T, tail_lin).reshape(V_NUM, H_DIM)
    out5 = _KERNEL(idx, lin_table)
    return out5.transpose(2, 4, 0, 1, 3).reshape(BATCH, FIELDS, H_DIM)


# speedup vs baseline: 3.1145x; 1.0012x over previous
import functools

import jax
import jax.numpy as jnp
from jax import lax
from jax.experimental import pallas as pl
from jax.experimental.pallas import tpu as pltpu
from jax.experimental.pallas import tpu_sc as plsc

V_NUM = 1000000
H_DIM = 32
BATCH = 16384
FIELDS = 26

_N = BATCH * FIELDS
_NW = 32
_PER_W = _N // _NW
_GRP = 128
_G_PER_W = _PER_W // _GRP
_GPC = 8
_CH = _GPC * _GRP
_NCH = _G_PER_W // _GPC
_NBUF = 3
_TCG = BATCH // _GRP


_VFULL = V_NUM // _GRP * _GRP
_NFG = _VFULL // _GRP
_VTAIL = V_NUM - _VFULL


def _make_detile_kernel():
    mesh = plsc.VectorSubcoreMesh(core_axis_name="c", subcore_axis_name="s")

    @functools.partial(
        pl.kernel,
        mesh=mesh,
        out_type=jax.ShapeDtypeStruct((V_NUM * H_DIM,), jnp.float32),
        compiler_params=pltpu.CompilerParams(
            use_tc_tiling_on_sc=True, needs_layout_passes=False),
        scratch_types=(
            pltpu.VMEM((2 * H_DIM, _GRP), jnp.float32),
            pltpu.VMEM((2 * _GRP * H_DIM,), jnp.float32),
            pltpu.SemaphoreType.DMA((2,)),
            pltpu.SemaphoreType.DMA((2,)),
        ),
    )
    def detile_kernel(tab_t_hbm, tail_hbm, out_hbm, tsrc, tdst, rsem, wsem):
        wid = lax.axis_index("s") * 2 + lax.axis_index("c")
        cnt = jnp.where(wid < 4, _NFG // _NW + 1, _NFG // _NW)
        iota = lax.iota(jnp.int32, 16)
        rots = [lax.rem(iota + d, 16) for d in range(16)]
        rot32s = [r * H_DIM + iota for r in rots]

        def r_copies(i, s):
            g = i * _NW + wid
            for tr in range(H_DIM // 8):
                yield pltpu.make_async_copy(
                    tab_t_hbm.at[pl.ds(8 * tr, 8), pl.ds(g * _GRP, _GRP)],
                    tsrc.at[pl.ds(s * H_DIM + 8 * tr, 8)],
                    rsem.at[s])

        def transpose_block(s, l0, nl):
            for lb in range(0, nl, 16):
                for k in range(H_DIM // 16):
                    gidx0 = s * H_DIM + 16 * k + iota
                    sbase = s * (_GRP * H_DIM) + (l0 + lb) * H_DIM + 16 * k
                    for d in range(16):
                        vec = plsc.load_gather(
                            tsrc, [gidx0, l0 + lb + rots[d]])
                        plsc.store_scatter(tdst, [rot32s[d] + sbase], vec)

        def w_copy(i, s):
            g = i * _NW + wid
            return pltpu.make_async_copy(
                tdst.at[pl.ds(s * (_GRP * H_DIM), _GRP * H_DIM)],
                out_hbm.at[pl.ds(g * _GRP * H_DIM, _GRP * H_DIM)],
                wsem.at[s])

        for c in r_copies(0, 0):
            c.start()

        def body(i, _):
            s = lax.rem(i, 2)
            for c in r_copies(i, s):
                c.wait()

            @pl.when(i + 1 < cnt)
            def _():
                for c in r_copies(i + 1, 1 - s):
                    c.start()

            @pl.when(i >= 2)
            def _():
                w_copy(i - 2, s).wait()

            transpose_block(s, 0, _GRP)
            w_copy(i, s).start()
            return _

        lax.fori_loop(0, cnt, body, None, unroll=False)
        for back in range(2):
            @pl.when(cnt > back)
            def _():
                i = cnt - 1 - back
                w_copy(i, lax.rem(i, 2)).wait()

        @pl.when(wid == _NW - 1)
        def _():
            pltpu.sync_copy(tail_hbm, tdst.at[pl.ds(0, _VTAIL * H_DIM)])
            pltpu.sync_copy(
                tdst.at[pl.ds(0, _VTAIL * H_DIM)],
                out_hbm.at[pl.ds(_VFULL * H_DIM, _VTAIL * H_DIM)])

    return detile_kernel


def _make_kernel():
    mesh = plsc.VectorSubcoreMesh(core_axis_name="c", subcore_axis_name="s")

    @functools.partial(
        pl.kernel,
        mesh=mesh,
        out_type=jax.ShapeDtypeStruct((FIELDS, H_DIM // 8, _TCG, 8, _GRP),
                                      jnp.float32),
        compiler_params=pltpu.CompilerParams(
            use_tc_tiling_on_sc=False, needs_layout_passes=False),
        scratch_types=(
            pltpu.VMEM((_PER_W,), jnp.int32),
            pltpu.VMEM((_NBUF * _CH, H_DIM), jnp.float32),
            pltpu.VMEM((2, H_DIM, 129), jnp.float32),
            pltpu.SemaphoreType.DMA((_NBUF,)),
            pltpu.SemaphoreType.DMA((2,)),
        ),
    )
    def gather_kernel(idx_hbm, table_hbm, out_hbm, idx_v, rows, tbuf,
                      gsem, wsem):
        wid = lax.axis_index("s") * 2 + lax.axis_index("c")
        base = wid * _PER_W
        g_base = wid * _G_PER_W
        pltpu.sync_copy(idx_hbm.at[pl.ds(base, _PER_W)], idx_v)

        def g_start(ch, slot):
            pltpu.async_copy(
                table_hbm.at[idx_v.at[pl.ds(ch * _CH, _CH)]],
                rows.at[pl.ds(slot * _CH, _CH)],
                gsem.at[slot])

        def g_wait(ch, slot):
            pltpu.make_async_copy(
                table_hbm.at[idx_v.at[pl.ds(ch * _CH, _CH)]],
                rows.at[pl.ds(slot * _CH, _CH)],
                gsem.at[slot]).wait()

        def t_writes(gg, p, do_wait):
            f = gg // _TCG
            tc = gg % _TCG
            for tr in range(H_DIM // 8):
                cpy = pltpu.make_async_copy(
                    tbuf.at[p, pl.ds(8 * tr, 8), pl.ds(0, _GRP)],
                    out_hbm.at[f, tr, tc],
                    wsem.at[p])
                if do_wait:
                    cpy.wait()
                else:
                    cpy.start()

        iota = lax.iota(jnp.int32, 16)

        for slot in range(_NBUF):
            g_start(slot, slot)

        def chunk_body(ch, _):
            slot = lax.rem(ch, _NBUF)
            g_wait(ch, slot)

            def group_body(g8, _):
                gl = ch * _GPC + g8
                gg = g_base + gl
                p = lax.rem(gl, 2)

                @pl.when(gl >= 2)
                def _():
                    t_writes(gg - 2, p, True)

                row0 = slot * _CH + g8 * _GRP
                for l in range(_GRP):
                    for k in range(H_DIM // 16):
                        vec = rows[row0 + l, pl.ds(16 * k, 16)]
                        plsc.store_scatter(
                            tbuf.at[p],
                            [16 * k + iota, jnp.full((16,), l, jnp.int32)],
                            vec)
                t_writes(gg, p, False)
                return _

            lax.fori_loop(0, _GPC, group_body, None, unroll=False)

            @pl.when(ch + _NBUF < _NCH)
            def _():
                g_start(ch + _NBUF, slot)
            return _

        lax.fori_loop(0, _NCH, chunk_body, None, unroll=False)

        for tail in range(2):
            gl = _G_PER_W - 2 + tail
            t_writes(g_base + gl, gl % 2, True)

    return gather_kernel


_KERNEL = _make_kernel()
_DETILE = _make_detile_kernel()


@jax.jit
def kernel(x, table):
    idx = x.T.reshape(-1).astype(jnp.int32)
    tail_lin = table[_VFULL:].reshape(-1)
    lin_table = _DETILE(table.T, tail_lin).reshape(V_NUM, H_DIM)
    out5 = _KERNEL(idx, lin_table)
    return out5.transpose(2, 4, 0, 1, 3).reshape(BATCH, FIELDS, H_DIM)
